# Initial kernel scaffold; baseline (speedup 1.0000x reference)
import jax
import jax.numpy as jnp
from jax.experimental import pallas as pl


def kernel(x, emb, W_ih0, W_hh0, b_ih0, b_hh0, W_ih1, W_hh1, b_ih1, b_hh1, fc1_w, fc1_b, fc2_w, fc2_b):
    raise NotImplementedError("write your pallas kernel here")


# speedup vs baseline: 15.4448x; 15.4448x over previous
import functools

import jax
import jax.numpy as jnp
from jax import lax
from jax.experimental import pallas as pl
from jax.experimental.pallas import tpu as pltpu
from jax.experimental.pallas import tpu_sc as plsc


def _sc_gather(emb, idx, chunk=128):
    n = idx.shape[0]
    d = emb.shape[1]
    info = plsc.get_sparse_core_info()
    nw = info.num_cores * info.num_subcores
    per_w = n // nw
    assert per_w * nw == n and per_w % chunk == 0
    n_chunks = per_w // chunk

    mesh = plsc.VectorSubcoreMesh(core_axis_name="c", subcore_axis_name="s")

    @functools.partial(
        pl.kernel,
        mesh=mesh,
        out_type=jax.ShapeDtypeStruct((n, d), jnp.float32),
        scratch_types=[
            pltpu.VMEM((chunk,), jnp.int32),
            pltpu.VMEM((chunk, d), jnp.float32),
            pltpu.SemaphoreType.DMA,
        ],
    )
    def gather_kernel(emb_hbm, idx_hbm, out_hbm, idx_v, rows_v, sem):
        wid = lax.axis_index("s") * info.num_cores + lax.axis_index("c")
        base = wid * per_w

        def body(i, carry):
            off = base + i * chunk
            pltpu.sync_copy(idx_hbm.at[pl.ds(off, chunk)], idx_v)
            pltpu.async_copy(emb_hbm.at[idx_v], rows_v, sem).wait()
            pltpu.sync_copy(rows_v, out_hbm.at[pl.ds(off, chunk)])
            return carry

        lax.fori_loop(0, n_chunks, body, 0)

    return gather_kernel(emb, idx)


def _rnn_step(x_ref, wih0_ref, whh0_ref, b0_ref, wih1_ref, whh1_ref, b1_ref,
              fc1w_ref, fc1b_ref, fc2w_ref, fc2b_ref, out_ref, h1_ref, h2_ref,
              *, n_steps):
    t = pl.program_id(0)

    @pl.when(t == 0)
    def _():
        h1_ref[...] = jnp.zeros_like(h1_ref)
        h2_ref[...] = jnp.zeros_like(h2_ref)

    x_t = x_ref[0]
    a1 = (jnp.dot(x_t, wih0_ref[...], preferred_element_type=jnp.float32)
          + jnp.dot(h1_ref[...], whh0_ref[...], preferred_element_type=jnp.float32)
          + b0_ref[...])
    h1 = jnp.tanh(a1)
    h1_ref[...] = h1
    a2 = (jnp.dot(h1, wih1_ref[...], preferred_element_type=jnp.float32)
          + jnp.dot(h2_ref[...], whh1_ref[...], preferred_element_type=jnp.float32)
          + b1_ref[...])
    h2 = jnp.tanh(a2)
    h2_ref[...] = h2

    @pl.when(t == n_steps - 1)
    def _():
        r = jnp.maximum(
            jnp.dot(h2, fc1w_ref[...], preferred_element_type=jnp.float32)
            + fc1b_ref[...], 0.0)
        out_ref[...] = (jnp.dot(r, fc2w_ref[...],
                                preferred_element_type=jnp.float32)
                        + fc2b_ref[...])


def _tc_rnn(xemb, wih0, whh0, b0, wih1, whh1, b1, fc1w, fc1b, fc2w, fc2b):
    l, b, e = xemb.shape
    hid = whh0.shape[0]
    f1 = fc1w.shape[1]

    full = lambda shape: pl.BlockSpec(shape, lambda t: (0,) * len(shape))
    return pl.pallas_call(
        functools.partial(_rnn_step, n_steps=l),
        grid=(l,),
        in_specs=[
            pl.BlockSpec((1, b, e), lambda t: (t, 0, 0)),
            full((e, hid)), full((hid, hid)), full((1, hid)),
            full((hid, hid)), full((hid, hid)), full((1, hid)),
            full((hid, f1)), full((1, f1)), full((f1, 1)), full((1, 1)),
        ],
        out_specs=pl.BlockSpec((b, 1), lambda t: (0, 0)),
        out_shape=jax.ShapeDtypeStruct((b, 1), jnp.float32),
        scratch_shapes=[
            pltpu.VMEM((b, hid), jnp.float32),
            pltpu.VMEM((b, hid), jnp.float32),
        ],
        compiler_params=pltpu.CompilerParams(
            dimension_semantics=("arbitrary",)),
    )(xemb, wih0, whh0, b0, wih1, whh1, b1, fc1w, fc1b, fc2w, fc2b)


def kernel(x, emb, W_ih0, W_hh0, b_ih0, b_hh0, W_ih1, W_hh1, b_ih1, b_hh1,
           fc1_w, fc1_b, fc2_w, fc2_b):
    b, l = x.shape
    e = emb.shape[1]
    idx = x.astype(jnp.int32).T.reshape(-1)
    xemb = _sc_gather(emb, idx).reshape(l, b, e)
    out = _tc_rnn(
        xemb,
        W_ih0.T, W_hh0.T, (b_ih0 + b_hh0).reshape(1, -1),
        W_ih1.T, W_hh1.T, (b_ih1 + b_hh1).reshape(1, -1),
        fc1_w.T, fc1_b.reshape(1, -1), fc2_w.T, fc2_b.reshape(1, -1),
    )
    return out


# speedup vs baseline: 22.6450x; 1.4662x over previous
import functools

import jax
import jax.numpy as jnp
from jax import lax
from jax.experimental import pallas as pl
from jax.experimental.pallas import tpu as pltpu
from jax.experimental.pallas import tpu_sc as plsc


def _sc_gather(emb, idx, chunk=128):
    n = idx.shape[0]
    d = emb.shape[1]
    info = plsc.get_sparse_core_info()
    nw = info.num_cores * info.num_subcores
    per_w = n // nw
    assert per_w * nw == n and per_w % chunk == 0
    n_chunks = per_w // chunk

    mesh = plsc.VectorSubcoreMesh(core_axis_name="c", subcore_axis_name="s")

    @functools.partial(
        pl.kernel,
        mesh=mesh,
        out_type=jax.ShapeDtypeStruct((n, d), jnp.float32),
        scratch_types=[
            pltpu.VMEM((chunk,), jnp.int32),
            pltpu.VMEM((chunk, d), jnp.float32),
            pltpu.SemaphoreType.DMA,
        ],
    )
    def gather_kernel(emb_hbm, idx_hbm, out_hbm, idx_v, rows_v, sem):
        wid = lax.axis_index("s") * info.num_cores + lax.axis_index("c")
        base = wid * per_w

        def body(i, carry):
            off = base + i * chunk
            pltpu.sync_copy(idx_hbm.at[pl.ds(off, chunk)], idx_v)
            pltpu.async_copy(emb_hbm.at[idx_v], rows_v, sem).wait()
            pltpu.sync_copy(rows_v, out_hbm.at[pl.ds(off, chunk)])
            return carry

        lax.fori_loop(0, n_chunks, body, 0)

    return gather_kernel(emb, idx)


def _rnn_chunk_step(x_ref, h1i_ref, h2i_ref, wih0_ref, whh0_ref, b0_ref,
                    wih1_ref, whh1_ref, b1_ref, fc1w_ref, fc1b_ref,
                    fc2w_ref, fc2b_ref, *out_refs, n_steps, final):
    t = pl.program_id(0)
    if final:
        out_ref, h1_ref, h2_ref = out_refs[0], out_refs[1], out_refs[2]
    else:
        h1_ref, h2_ref = out_refs[0], out_refs[1]

    @pl.when(t == 0)
    def _():
        h1_ref[...] = h1i_ref[...]
        h2_ref[...] = h2i_ref[...]

    x_t = x_ref[0]
    a1 = (jnp.dot(x_t, wih0_ref[...], preferred_element_type=jnp.float32)
          + jnp.dot(h1_ref[...], whh0_ref[...], preferred_element_type=jnp.float32)
          + b0_ref[...])
    h1 = jnp.tanh(a1)
    h1_ref[...] = h1
    a2 = (jnp.dot(h1, wih1_ref[...], preferred_element_type=jnp.float32)
          + jnp.dot(h2_ref[...], whh1_ref[...], preferred_element_type=jnp.float32)
          + b1_ref[...])
    h2 = jnp.tanh(a2)
    h2_ref[...] = h2

    if final:
        @pl.when(t == n_steps - 1)
        def _():
            r = jnp.maximum(
                jnp.dot(h2, fc1w_ref[...], preferred_element_type=jnp.float32)
                + fc1b_ref[...], 0.0)
            out_ref[...] = (jnp.dot(r, fc2w_ref[...],
                                    preferred_element_type=jnp.float32)
                            + fc2b_ref[...])


def _tc_rnn_chunk(xemb, h1_in, h2_in, weights, final):
    l, b, e = xemb.shape
    hid = h1_in.shape[1]
    f1 = weights[6].shape[1]

    full = lambda shape: pl.BlockSpec(shape, lambda t: (0,) * len(shape))
    hspec = full((b, hid))
    out_specs = [hspec, hspec]
    out_shape = [jax.ShapeDtypeStruct((b, hid), jnp.float32),
                 jax.ShapeDtypeStruct((b, hid), jnp.float32)]
    if final:
        out_specs = [full((b, 1))] + out_specs
        out_shape = [jax.ShapeDtypeStruct((b, 1), jnp.float32)] + out_shape
    return pl.pallas_call(
        functools.partial(_rnn_chunk_step, n_steps=l, final=final),
        grid=(l,),
        in_specs=[
            pl.BlockSpec((1, b, e), lambda t: (t, 0, 0)),
            hspec, hspec,
            full((e, hid)), full((hid, hid)), full((1, hid)),
            full((hid, hid)), full((hid, hid)), full((1, hid)),
            full((hid, f1)), full((1, f1)), full((f1, 1)), full((1, 1)),
        ],
        out_specs=out_specs,
        out_shape=out_shape,
        compiler_params=pltpu.CompilerParams(
            dimension_semantics=("arbitrary",)),
    )(xemb, h1_in, h2_in, *weights)


_N_CHUNKS = 4


def kernel(x, emb, W_ih0, W_hh0, b_ih0, b_hh0, W_ih1, W_hh1, b_ih1, b_hh1,
           fc1_w, fc1_b, fc2_w, fc2_b):
    b, l = x.shape
    e = emb.shape[1]
    hid = W_hh0.shape[0]
    idx = x.astype(jnp.int32).T.reshape(-1)
    lc = l // _N_CHUNKS
    gs = [_sc_gather(emb, lax.slice_in_dim(idx, c * lc * b, (c + 1) * lc * b))
          .reshape(lc, b, e) for c in range(_N_CHUNKS)]
    weights = (W_ih0.T, W_hh0.T, (b_ih0 + b_hh0).reshape(1, -1),
               W_ih1.T, W_hh1.T, (b_ih1 + b_hh1).reshape(1, -1),
               fc1_w.T, fc1_b.reshape(1, -1), fc2_w.T, fc2_b.reshape(1, -1))
    h1 = jnp.zeros((b, hid), jnp.float32)
    h2 = jnp.zeros((b, hid), jnp.float32)
    for c in range(_N_CHUNKS - 1):
        h1, h2 = _tc_rnn_chunk(gs[c], h1, h2, weights, final=False)
    out, _, _ = _tc_rnn_chunk(gs[-1], h1, h2, weights, final=True)
    return out


# speedup vs baseline: 23.7751x; 1.0499x over previous
import functools

import jax
import jax.numpy as jnp
from jax import lax
from jax.experimental import pallas as pl
from jax.experimental.pallas import tpu as pltpu
from jax.experimental.pallas import tpu_sc as plsc


def _sc_gather(emb, idx, chunk=128):
    n = idx.shape[0]
    d = emb.shape[1]
    info = plsc.get_sparse_core_info()
    nw = info.num_cores * info.num_subcores
    per_w = n // nw
    assert per_w * nw == n and per_w % chunk == 0
    n_chunks = per_w // chunk

    mesh = plsc.VectorSubcoreMesh(core_axis_name="c", subcore_axis_name="s")

    @functools.partial(
        pl.kernel,
        mesh=mesh,
        out_type=jax.ShapeDtypeStruct((n, d), emb.dtype),
        scratch_types=[
            pltpu.VMEM((chunk,), jnp.int32),
            pltpu.VMEM((chunk, d), emb.dtype),
            pltpu.SemaphoreType.DMA,
        ],
    )
    def gather_kernel(emb_hbm, idx_hbm, out_hbm, idx_v, rows_v, sem):
        wid = lax.axis_index("s") * info.num_cores + lax.axis_index("c")
        base = wid * per_w

        def body(i, carry):
            off = base + i * chunk
            pltpu.sync_copy(idx_hbm.at[pl.ds(off, chunk)], idx_v)
            pltpu.async_copy(emb_hbm.at[idx_v], rows_v, sem).wait()
            pltpu.sync_copy(rows_v, out_hbm.at[pl.ds(off, chunk)])
            return carry

        lax.fori_loop(0, n_chunks, body, 0)

    return gather_kernel(emb, idx)


def _rnn_chunk_step(x_ref, h1i_ref, h2i_ref, wih0_ref, whh0_ref, b0_ref,
                    wih1_ref, whh1_ref, b1_ref, fc1w_ref, fc1b_ref,
                    fc2w_ref, fc2b_ref, *out_refs, n_steps, final):
    t = pl.program_id(0)
    if final:
        out_ref, h1_ref, h2_ref = out_refs[0], out_refs[1], out_refs[2]
    else:
        h1_ref, h2_ref = out_refs[0], out_refs[1]

    @pl.when(t == 0)
    def _():
        h1_ref[...] = h1i_ref[...]
        h2_ref[...] = h2i_ref[...]

    x_t = x_ref[0].astype(jnp.float32)
    a1 = (jnp.dot(x_t, wih0_ref[...], preferred_element_type=jnp.float32)
          + jnp.dot(h1_ref[...], whh0_ref[...], preferred_element_type=jnp.float32)
          + b0_ref[...])
    h1 = jnp.tanh(a1)
    h1_ref[...] = h1
    a2 = (jnp.dot(h1, wih1_ref[...], preferred_element_type=jnp.float32)
          + jnp.dot(h2_ref[...], whh1_ref[...], preferred_element_type=jnp.float32)
          + b1_ref[...])
    h2 = jnp.tanh(a2)
    h2_ref[...] = h2

    if final:
        @pl.when(t == n_steps - 1)
        def _():
            r = jnp.maximum(
                jnp.dot(h2, fc1w_ref[...], preferred_element_type=jnp.float32)
                + fc1b_ref[...], 0.0)
            out_ref[...] = (jnp.dot(r, fc2w_ref[...],
                                    preferred_element_type=jnp.float32)
                            + fc2b_ref[...])


def _tc_rnn_chunk(xemb, h1_in, h2_in, weights, final):
    l, b, e = xemb.shape
    hid = h1_in.shape[1]
    f1 = weights[6].shape[1]

    full = lambda shape: pl.BlockSpec(shape, lambda t: (0,) * len(shape))
    hspec = full((b, hid))
    out_specs = [hspec, hspec]
    out_shape = [jax.ShapeDtypeStruct((b, hid), jnp.float32),
                 jax.ShapeDtypeStruct((b, hid), jnp.float32)]
    if final:
        out_specs = [full((b, 1))] + out_specs
        out_shape = [jax.ShapeDtypeStruct((b, 1), jnp.float32)] + out_shape
    return pl.pallas_call(
        functools.partial(_rnn_chunk_step, n_steps=l, final=final),
        grid=(l,),
        in_specs=[
            pl.BlockSpec((1, b, e), lambda t: (t, 0, 0)),
            hspec, hspec,
            full((e, hid)), full((hid, hid)), full((1, hid)),
            full((hid, hid)), full((hid, hid)), full((1, hid)),
            full((hid, f1)), full((1, f1)), full((f1, 1)), full((1, 1)),
        ],
        out_specs=out_specs,
        out_shape=out_shape,
        compiler_params=pltpu.CompilerParams(
            dimension_semantics=("arbitrary",)),
    )(xemb, h1_in, h2_in, *weights)


_N_CHUNKS = 8


def kernel(x, emb, W_ih0, W_hh0, b_ih0, b_hh0, W_ih1, W_hh1, b_ih1, b_hh1,
           fc1_w, fc1_b, fc2_w, fc2_b):
    b, l = x.shape
    e = emb.shape[1]
    hid = W_hh0.shape[0]
    idx = x.astype(jnp.int32).T.reshape(-1)
    lc = l // _N_CHUNKS
    gs = [_sc_gather(emb, lax.slice_in_dim(idx, c * lc * b,
                                           (c + 1) * lc * b))
          .reshape(lc, b, e)
          for c in range(_N_CHUNKS)]
    weights = (W_ih0.T, W_hh0.T, (b_ih0 + b_hh0).reshape(1, -1),
               W_ih1.T, W_hh1.T, (b_ih1 + b_hh1).reshape(1, -1),
               fc1_w.T, fc1_b.reshape(1, -1), fc2_w.T, fc2_b.reshape(1, -1))
    h1 = jnp.zeros((b, hid), jnp.float32)
    h2 = jnp.zeros((b, hid), jnp.float32)
    for c in range(_N_CHUNKS - 1):
        h1, h2 = _tc_rnn_chunk(gs[c], h1, h2, weights, final=False)
    out, _, _ = _tc_rnn_chunk(gs[-1], h1, h2, weights, final=True)
    return out
